# baseline (speedup 1.0000x reference)
import functools
import math

import jax
import jax.numpy as jnp
import numpy as np
from jax import lax
from jax.experimental import pallas as pl
from jax.experimental.pallas import tpu as pltpu
from jax.experimental.pallas import tpu_sc as plsc

VOCAB = 100000
D = 512
BATCH = 16
SEQ = 2048
L = 16
NC = 2
NS = 16
NW = NC * NS
N = BATCH * SEQ
P_PER_W = SEQ // NW
C = 32
N_CHUNKS = BATCH * P_PER_W // C
SCALE = math.sqrt(float(D))


def _pe_table():
    pos = np.arange(SEQ, dtype=np.float32).reshape(-1, 1)
    i = np.arange(D, dtype=np.float32)
    rads = pos / np.power(10000.0, 2.0 * np.floor(i / 2.0) / D)
    pe = np.zeros((SEQ, D), dtype=np.float32)
    pe[:, 0::2] = np.sin(rads[:, 0::2])
    pe[:, 1::2] = np.cos(rads[:, 1::2])
    return pe


_PE = _pe_table()


@functools.partial(
    pl.kernel,
    out_type=jax.ShapeDtypeStruct((N, D), jnp.float32),
    mesh=plsc.VectorSubcoreMesh(core_axis_name="c", subcore_axis_name="s"),
    scratch_types=[
        pltpu.VMEM((BATCH, 2 * P_PER_W), jnp.int32),
        pltpu.VMEM((P_PER_W, D), jnp.float32),
        pltpu.VMEM((C, D), jnp.float32),
        pltpu.VMEM((C, D), jnp.float32),
        pltpu.VMEM((C, D), jnp.float32),
        pltpu.VMEM((C, D), jnp.float32),
        pltpu.SemaphoreType.DMA,
        pltpu.SemaphoreType.DMA,
        pltpu.SemaphoreType.DMA,
        pltpu.SemaphoreType.DMA,
    ],
)
def _emb_lookup(table_hbm, idx_hbm, pe_hbm, out_hbm,
                idx_all, pe_v, rows0, rows1, ost0, ost1,
                g0, g1, o0, o1):
    rows = (rows0, rows1)
    ost = (ost0, ost1)
    gsm = (g0, g1)
    osm = (o0, o1)

    wid = lax.axis_index("s") * NC + lax.axis_index("c")
    pos0 = wid * P_PER_W

    half = lax.rem(wid, 2) * P_PER_W
    pltpu.sync_copy(pe_hbm.at[pl.ds(pos0, P_PER_W)], pe_v)
    pltpu.sync_copy(idx_hbm.at[:, pl.ds((wid // 2) * (2 * P_PER_W),
                                        2 * P_PER_W)], idx_all)

    def idx_slice(bj, h):
        return idx_all.at[bj, pl.ds(half + h * C, C)]

    def out_slice(bj, h):
        return out_hbm.at[pl.ds(bj * SEQ + pos0 + h * C, C)]

    pltpu.async_copy(table_hbm.at[idx_slice(0, 0)], rows[0], gsm[0])

    @pl.loop(0, BATCH)
    def _batch(bj):
        for h in (0, 1):
            j = 2 * bj + h

            if h == 0:
                pltpu.async_copy(table_hbm.at[idx_slice(bj, 1)],
                                 rows[1], gsm[1])
            else:
                @pl.when(bj + 1 < BATCH)
                def _():
                    pltpu.async_copy(table_hbm.at[idx_slice(bj + 1, 0)],
                                     rows[0], gsm[0])

            pltpu.make_async_copy(table_hbm.at[idx_slice(bj, h)],
                                  rows[h], gsm[h]).wait()

            @pl.when(j >= 2)
            def _():
                pltpu.make_async_copy(ost[h], out_slice(bj - 1, h),
                                      osm[h]).wait()

            @plsc.parallel_loop(0, C, unroll=1)
            def row_body(r, _h=h):
                @plsc.parallel_loop(0, D // L, unroll=4)
                def col_body(c, _h=_h, r=r):
                    sl = pl.ds(c * L, L)
                    ost[_h][r, sl] = (rows[_h][r, sl] * SCALE
                                      + pe_v[_h * C + r, sl])

            pltpu.async_copy(ost[h], out_slice(bj, h), osm[h])

    pltpu.make_async_copy(ost[0], out_slice(BATCH - 1, 0), osm[0]).wait()
    pltpu.make_async_copy(ost[1], out_slice(BATCH - 1, 1), osm[1]).wait()


def kernel(x, table):
    pe = jnp.asarray(_PE)
    out = _emb_lookup(table, x, pe)
    return out.reshape(BATCH, SEQ, D)


# speedup vs baseline: 1.0397x; 1.0397x over previous
import functools
import math

import jax
import jax.numpy as jnp
import numpy as np
from jax import lax
from jax.experimental import pallas as pl
from jax.experimental.pallas import tpu as pltpu
from jax.experimental.pallas import tpu_sc as plsc

VOCAB = 100000
D = 512
BATCH = 16
SEQ = 2048
L = 16
NC = 2
NS = 16
NW = NC * NS
N = BATCH * SEQ
P_PER_W = SEQ // NW
C = 32
N_CHUNKS = BATCH * P_PER_W // C
SCALE = math.sqrt(float(D))


def _pe_table():
    pos = np.arange(SEQ, dtype=np.float32).reshape(-1, 1)
    i = np.arange(D, dtype=np.float32)
    rads = pos / np.power(10000.0, 2.0 * np.floor(i / 2.0) / D)
    pe = np.zeros((SEQ, D), dtype=np.float32)
    pe[:, 0::2] = np.sin(rads[:, 0::2])
    pe[:, 1::2] = np.cos(rads[:, 1::2])
    return pe


_PE = _pe_table()


@functools.partial(
    pl.kernel,
    out_type=jax.ShapeDtypeStruct((N, D), jnp.float32),
    mesh=plsc.VectorSubcoreMesh(core_axis_name="c", subcore_axis_name="s"),
    scratch_types=[
        pltpu.VMEM((BATCH, 2 * P_PER_W), jnp.int32),
        pltpu.VMEM((P_PER_W, D), jnp.float32),
        pltpu.VMEM((C, D), jnp.float32),
        pltpu.VMEM((C, D), jnp.float32),
        pltpu.VMEM((C, D), jnp.float32),
        pltpu.VMEM((C, D), jnp.float32),
        pltpu.SemaphoreType.DMA,
        pltpu.SemaphoreType.DMA,
        pltpu.SemaphoreType.DMA,
        pltpu.SemaphoreType.DMA,
    ],
)
def _emb_lookup(table_hbm, idx_hbm, pe_hbm, out_hbm,
                idx_all, pe_v, rows0, rows1, ost0, ost1,
                g0, g1, o0, o1):
    rows = (rows0, rows1)
    ost = (ost0, ost1)
    gsm = (g0, g1)
    osm = (o0, o1)

    wid = lax.axis_index("s") * NC + lax.axis_index("c")
    pos0 = wid * P_PER_W

    half = lax.rem(wid, 2) * P_PER_W
    pltpu.sync_copy(pe_hbm.at[pl.ds(pos0, P_PER_W)], pe_v)
    pltpu.sync_copy(idx_hbm.at[:, pl.ds((wid // 2) * (2 * P_PER_W),
                                        2 * P_PER_W)], idx_all)

    def idx_slice(bj, h):
        return idx_all.at[bj, pl.ds(half + h * C, C)]

    def out_slice(bj, h):
        return out_hbm.at[pl.ds(bj * SEQ + pos0 + h * C, C)]

    pltpu.async_copy(table_hbm.at[idx_slice(0, 0)], rows[0], gsm[0])

    @pl.loop(0, BATCH)
    def _batch(bj):
        for h in (0, 1):
            j = 2 * bj + h

            if h == 0:
                pltpu.async_copy(table_hbm.at[idx_slice(bj, 1)],
                                 rows[1], gsm[1])
            else:
                @pl.when(bj + 1 < BATCH)
                def _():
                    pltpu.async_copy(table_hbm.at[idx_slice(bj + 1, 0)],
                                     rows[0], gsm[0])

            pltpu.make_async_copy(table_hbm.at[idx_slice(bj, h)],
                                  rows[h], gsm[h]).wait()

            @pl.when(j >= 2)
            def _():
                pltpu.make_async_copy(ost[h], out_slice(bj - 1, h),
                                      osm[h]).wait()

            @plsc.parallel_loop(0, C, unroll=1)
            def row_body(r, _h=h):
                for c in range(D // L):
                    sl = pl.ds(c * L, L)
                    ost[_h][r, sl] = (rows[_h][r, sl] * SCALE
                                      + pe_v[_h * C + r, sl])

            pltpu.async_copy(ost[h], out_slice(bj, h), osm[h])

    pltpu.make_async_copy(ost[0], out_slice(BATCH - 1, 0), osm[0]).wait()
    pltpu.make_async_copy(ost[1], out_slice(BATCH - 1, 1), osm[1]).wait()


def kernel(x, table):
    pe = jnp.asarray(_PE)
    out = _emb_lookup(table, x, pe)
    return out.reshape(BATCH, SEQ, D)
